# baseline (speedup 1.0000x reference)
import functools

import jax
import jax.numpy as jnp
from jax import lax
from jax.experimental import pallas as pl
from jax.experimental.pallas import tpu as pltpu
from jax.experimental.pallas import tpu_sc as plsc

D = 64
B = 4096
K = 128
GAMMA = 24.0
ALPHA = 0.02
NC = 2
NS = 16
NW = NC * NS
BQ = B // NW
L = 16
NG = K // L
QG = BQ // L
DG = D // L


def _sc_body(ew, ow, ans, tm_t, ta_t, sm_t, sa_t, w_hbm,
             hid_hbm, rid_hbm, pid_hbm, nid_hbm, out_hbm,
             hid_v, rid_v, pid_v, nid_v, w_v, eidx_v,
             cen_v, box_v, tm_v, ta_v, pr_v,
             na_v, nb_v, nc_v, nd_v, out_v, sem0, sem1, sem2, sem3):
  wid = lax.axis_index("s") * NC + lax.axis_index("c")
  base = wid * BQ
  iota = lax.iota(jnp.int32, L)

  pltpu.sync_copy(hid_hbm.at[pl.ds(base, BQ)], hid_v)
  pltpu.sync_copy(rid_hbm.at[pl.ds(base, BQ)], rid_v)
  pltpu.sync_copy(pid_hbm.at[pl.ds(base, BQ)], pid_v)
  pltpu.sync_copy(w_hbm.at[pl.ds(base, BQ)], w_v)
  pltpu.sync_copy(nid_hbm.at[pl.ds(base, BQ)], nid_v)

  d2 = pltpu.async_copy(tm_t.at[rid_v], tm_v, sem0)
  d3 = pltpu.async_copy(ta_t.at[rid_v], ta_v, sem0)
  d6 = pltpu.async_copy(ans.at[pid_v], pr_v, sem0)

  def _mk_idx(q, _):
    hq = plsc.load_gather(hid_v, [jnp.full((L,), q, jnp.int32)])
    for jg in range(DG):
      jv = (iota + jg * L) * 1000000
      eidx_v[pl.ds(q * D + jg * L, L)] = hq + jv
    return 0

  lax.fori_loop(0, BQ, _mk_idx, 0)
  for d in (d2, d3, d6):
    d.wait()

  def _egather(c0, _):
    ds_ = []
    for cc in range(8):
      sl = pl.ds((8 * c0 + cc) * 128, 128)
      ds_.append(pltpu.async_copy(ew.at[eidx_v.at[sl]], cen_v.at[sl], sem1))
      ds_.append(pltpu.async_copy(ow.at[eidx_v.at[sl]], box_v.at[sl], sem2))
    for d in ds_:
      d.wait()
    return 0

  lax.fori_loop(0, BQ * D // (128 * 8), _egather, 0)

  def _mk_center(b, _):
    for g in range(DG):
      sl = pl.ds(b * D + g * L, L)
      cen_v[sl] = (cen_v[sl] * tm_v[b, pl.ds(g * L, L)]
                   + ta_v[b, pl.ds(g * L, L)])
    return 0

  lax.fori_loop(0, BQ, _mk_center, 0)

  d4 = pltpu.async_copy(sm_t.at[rid_v], tm_v, sem0)
  d5 = pltpu.async_copy(sa_t.at[rid_v], ta_v, sem0)
  d4.wait()
  d5.wait()

  def _mk_box(b, _):
    for g in range(DG):
      sl = pl.ds(b * D + g * L, L)
      box_v[sl] = jnp.abs(box_v[sl] * tm_v[b, pl.ds(g * L, L)]
                          + ta_v[b, pl.ds(g * L, L)])
    return 0

  lax.fori_loop(0, BQ, _mk_box, 0)

  one_m_alpha = jnp.float32(1.0 - ALPHA)
  m15 = iota == (L - 1)

  def _row_logit(a_ref, row, cvec, ovec, wspl):
    u = None
    for j in range(DG):
      sl = pl.ds(j * L, L)
      dv = jnp.abs(a_ref[row, sl] - cvec[j])
      t = one_m_alpha * jnp.minimum(dv, ovec[j]) - dv
      u = t if u is None else u + t
    for sh in (8, 4, 2, 1):
      u = u + u.at[iota ^ sh].get(mode="promise_in_bounds")
    return (u + GAMMA) * wspl

  def _pos_q(b, _):
    bsplat = jnp.full((L,), b, jnp.int32)
    wspl = plsc.load_gather(w_v, [bsplat])
    cvec = [cen_v[pl.ds(b * D + j * L, L)] for j in range(DG)]
    ovec = [box_v[pl.ds(b * D + j * L, L)] for j in range(DG)]
    z = _row_logit(pr_v, b, cvec, ovec, wspl)
    plsc.store_scatter(out_v, [bsplat, jnp.zeros((L,), jnp.int32)], z,
                       mask=m15)
    return 0

  lax.fori_loop(0, BQ, _pos_q, 0)

  def _neg_compute(b, buf):
    bsplat = jnp.full((L,), b, jnp.int32)
    wspl = plsc.load_gather(w_v, [bsplat])
    cvec = [cen_v[pl.ds(b * D + j * L, L)] for j in range(DG)]
    ovec = [box_v[pl.ds(b * D + j * L, L)] for j in range(DG)]

    def _rows(i, _):
      for u in range(4):
        r = 4 * i + u
        z = _row_logit(buf, r, cvec, ovec, wspl)
        plsc.store_scatter(out_v, [bsplat, jnp.full((L,), 1 + r, jnp.int32)],
                           z, mask=m15)
      return 0

    lax.fori_loop(0, K // 4, _rows, 0)

  bufs = (na_v, nb_v, nc_v, nd_v)
  sems = (sem0, sem1, sem2, sem3)
  for u in range(3):
    pltpu.async_copy(ans.at[nid_v.at[u]], bufs[u], sems[u])

  def _quad(i, _):
    b0 = 4 * i
    for u in range(4):
      b = b0 + u
      pltpu.make_async_copy(ans.at[nid_v.at[b]], bufs[u], sems[u]).wait()

      @pl.when(b + 3 < BQ)
      def _():
        pltpu.async_copy(ans.at[nid_v.at[b + 3]], bufs[(u + 3) % 4],
                         sems[(u + 3) % 4])

      _neg_compute(b, bufs[u])
    return 0

  lax.fori_loop(0, BQ // 4, _quad, 0)

  pltpu.sync_copy(out_v, out_hbm.at[pl.ds(base, BQ)])


@jax.jit
def _run(ent, off_t, ans, tm_t, ta_t, sm_t, sa_t, w, hid, rid, pid, nid):
  mesh = plsc.VectorSubcoreMesh(core_axis_name="c", subcore_axis_name="s")
  f = functools.partial(
      pl.kernel,
      out_type=jax.ShapeDtypeStruct((B, 1 + K), jnp.float32),
      mesh=mesh,
      compiler_params=pltpu.CompilerParams(
          needs_layout_passes=False, use_tc_tiling_on_sc=False),
      scratch_types=[
          pltpu.VMEM((BQ,), jnp.int32),
          pltpu.VMEM((BQ,), jnp.int32),
          pltpu.VMEM((BQ,), jnp.int32),
          pltpu.VMEM((BQ, K), jnp.int32),
          pltpu.VMEM((BQ,), jnp.float32),
          pltpu.VMEM((BQ * D,), jnp.int32),
          pltpu.VMEM((BQ * D,), jnp.float32),
          pltpu.VMEM((BQ * D,), jnp.float32),
          pltpu.VMEM((BQ, D), jnp.float32),
          pltpu.VMEM((BQ, D), jnp.float32),
          pltpu.VMEM((BQ, D), jnp.float32),
          pltpu.VMEM((K, D), jnp.float32),
          pltpu.VMEM((K, D), jnp.float32),
          pltpu.VMEM((K, D), jnp.float32),
          pltpu.VMEM((K, D), jnp.float32),
          pltpu.VMEM((BQ, 1 + K), jnp.float32),
          pltpu.SemaphoreType.DMA,
          pltpu.SemaphoreType.DMA,
          pltpu.SemaphoreType.DMA,
          pltpu.SemaphoreType.DMA,
      ],
  )(_sc_body)
  return f(ent, off_t, ans, tm_t, ta_t, sm_t, sa_t, w, hid, rid, pid, nid)


def kernel(entity_embedding, offset_embedding, answer_embedding,
           translation_mul, translation_add, scaling_mul, scaling_add,
           subsampling_weight, head_ids, rel_ids, positive_sample,
           negative_sample):
  ew = jnp.ravel(entity_embedding.T)
  ow = jnp.ravel(offset_embedding.T)
  return _run(ew, ow, answer_embedding,
              translation_mul, translation_add, scaling_mul, scaling_add,
              subsampling_weight,
              head_ids.astype(jnp.int32), rel_ids.astype(jnp.int32),
              positive_sample.astype(jnp.int32),
              negative_sample.astype(jnp.int32))


# speedup vs baseline: 5.4742x; 5.4742x over previous
import functools

import jax
import jax.numpy as jnp
from jax import lax
from jax.experimental import pallas as pl
from jax.experimental import layout as jex_layout
from jax.experimental.pallas import tpu as pltpu
from jax.experimental.pallas import tpu_sc as plsc

D = 64
B = 4096
K = 128
GAMMA = 24.0
ALPHA = 0.02
NC = 2
NS = 16
NW = NC * NS
BQ = B // NW
L = 16
NG = K // L
QG = BQ // L
DG = D // L


def _sc_body(ent, off_t, ans, tm_t, ta_t, sm_t, sa_t, w_hbm,
             hid_hbm, rid_hbm, pid_hbm, nid_hbm, out_hbm,
             hid_v, rid_v, pid_v, nid_v, w_v,
             cen_v, box_v, tm_v, ta_v, sm_v, sa_v, pr_v,
             na_v, nb_v, nc_v, nd_v, out_v, sem0, sem1, sem2, sem3):
  wid = lax.axis_index("s") * NC + lax.axis_index("c")
  base = wid * BQ

  pltpu.sync_copy(hid_hbm.at[pl.ds(base, BQ)], hid_v)
  pltpu.sync_copy(rid_hbm.at[pl.ds(base, BQ)], rid_v)
  pltpu.sync_copy(pid_hbm.at[pl.ds(base, BQ)], pid_v)
  pltpu.sync_copy(w_hbm.at[pl.ds(base, BQ)], w_v)
  pltpu.sync_copy(nid_hbm.at[pl.ds(base, BQ)], nid_v)

  d0 = pltpu.async_copy(ent.at[hid_v], cen_v, sem0)
  d1 = pltpu.async_copy(off_t.at[hid_v], box_v, sem0)
  d2 = pltpu.async_copy(tm_t.at[rid_v], tm_v, sem0)
  d3 = pltpu.async_copy(ta_t.at[rid_v], ta_v, sem0)
  d4 = pltpu.async_copy(sm_t.at[rid_v], sm_v, sem0)
  d5 = pltpu.async_copy(sa_t.at[rid_v], sa_v, sem0)
  d6 = pltpu.async_copy(ans.at[pid_v], pr_v, sem0)
  for d in (d0, d1, d2, d3, d4, d5, d6):
    d.wait()

  def _mk_query(b, _):
    for g in range(DG):
      sl = pl.ds(g * L, L)
      e16 = cen_v[b, sl]
      cen_v[b, sl] = e16 * tm_v[b, sl] + ta_v[b, sl]
      o16 = box_v[b, sl]
      box_v[b, sl] = jnp.abs(o16 * sm_v[b, sl] + sa_v[b, sl])
    return 0

  lax.fori_loop(0, BQ, _mk_query, 0)

  iota = lax.iota(jnp.int32, L)
  one_m_alpha = jnp.float32(1.0 - ALPHA)
  m15 = iota == (L - 1)

  def _row_logit(a_ref, row, cvec, ovec, wspl):
    u = None
    for j in range(DG):
      sl = pl.ds(j * L, L)
      dv = jnp.abs(a_ref[row, sl] - cvec[j])
      t = one_m_alpha * jnp.minimum(dv, ovec[j]) - dv
      u = t if u is None else u + t
    for sh in (8, 4, 2, 1):
      u = u + u.at[iota ^ sh].get(mode="promise_in_bounds")
    return (u + GAMMA) * wspl

  def _pos_q(b, _):
    bsplat = jnp.full((L,), b, jnp.int32)
    wspl = plsc.load_gather(w_v, [bsplat])
    cvec = [cen_v[b, pl.ds(j * L, L)] for j in range(DG)]
    ovec = [box_v[b, pl.ds(j * L, L)] for j in range(DG)]
    z = _row_logit(pr_v, b, cvec, ovec, wspl)
    plsc.store_scatter(out_v, [bsplat, jnp.zeros((L,), jnp.int32)], z,
                       mask=m15)
    return 0

  lax.fori_loop(0, BQ, _pos_q, 0)

  def _neg_compute(b, buf):
    bsplat = jnp.full((L,), b, jnp.int32)
    wspl = plsc.load_gather(w_v, [bsplat])
    cvec = [cen_v[b, pl.ds(j * L, L)] for j in range(DG)]
    ovec = [box_v[b, pl.ds(j * L, L)] for j in range(DG)]

    def _rows(i, _):
      for u in range(4):
        r = 4 * i + u
        z = _row_logit(buf, r, cvec, ovec, wspl)
        plsc.store_scatter(out_v, [bsplat, jnp.full((L,), 1 + r, jnp.int32)],
                           z, mask=m15)
      return 0

    lax.fori_loop(0, K // 4, _rows, 0)

  bufs = (na_v, nb_v, nc_v, nd_v)
  sems = (sem0, sem1, sem2, sem3)
  for u in range(3):
    pltpu.async_copy(ans.at[nid_v.at[u]], bufs[u], sems[u])

  def _quad(i, _):
    b0 = 4 * i
    for u in range(4):
      b = b0 + u
      pltpu.make_async_copy(ans.at[nid_v.at[b]], bufs[u], sems[u]).wait()

      @pl.when(b + 3 < BQ)
      def _():
        pltpu.async_copy(ans.at[nid_v.at[b + 3]], bufs[(u + 3) % 4],
                         sems[(u + 3) % 4])

      _neg_compute(b, bufs[u])
    return 0

  lax.fori_loop(0, BQ // 4, _quad, 0)

  pltpu.sync_copy(out_v, out_hbm.at[pl.ds(base, BQ)])


@jax.jit
def _run(ent, off_t, ans, tm_t, ta_t, sm_t, sa_t, w, hid, rid, pid, nid):
  mesh = plsc.VectorSubcoreMesh(core_axis_name="c", subcore_axis_name="s")
  f = functools.partial(
      pl.kernel,
      out_type=jax.ShapeDtypeStruct((B, 1 + K), jnp.float32),
      mesh=mesh,
      compiler_params=pltpu.CompilerParams(
          needs_layout_passes=False, use_tc_tiling_on_sc=False),
      scratch_types=[
          pltpu.VMEM((BQ,), jnp.int32),
          pltpu.VMEM((BQ,), jnp.int32),
          pltpu.VMEM((BQ,), jnp.int32),
          pltpu.VMEM((BQ, K), jnp.int32),
          pltpu.VMEM((BQ,), jnp.float32),
          pltpu.VMEM((BQ, D), jnp.float32),
          pltpu.VMEM((BQ, D), jnp.float32),
          pltpu.VMEM((BQ, D), jnp.float32),
          pltpu.VMEM((BQ, D), jnp.float32),
          pltpu.VMEM((BQ, D), jnp.float32),
          pltpu.VMEM((BQ, D), jnp.float32),
          pltpu.VMEM((BQ, D), jnp.float32),
          pltpu.VMEM((K, D), jnp.float32),
          pltpu.VMEM((K, D), jnp.float32),
          pltpu.VMEM((K, D), jnp.float32),
          pltpu.VMEM((K, D), jnp.float32),
          pltpu.VMEM((BQ, 1 + K), jnp.float32),
          pltpu.SemaphoreType.DMA,
          pltpu.SemaphoreType.DMA,
          pltpu.SemaphoreType.DMA,
          pltpu.SemaphoreType.DMA,
      ],
  )(_sc_body)
  return f(ent, off_t, ans, tm_t, ta_t, sm_t, sa_t, w, hid, rid, pid, nid)


_ROWMAJOR_T8 = jex_layout.Layout(major_to_minor=(0, 1), tiling=((8,),))


def kernel(entity_embedding, offset_embedding, answer_embedding,
           translation_mul, translation_add, scaling_mul, scaling_add,
           subsampling_weight, head_ids, rel_ids, positive_sample,
           negative_sample):
  return _run(entity_embedding, offset_embedding, answer_embedding,
              translation_mul, translation_add, scaling_mul, scaling_add,
              subsampling_weight,
              head_ids.astype(jnp.int32), rel_ids.astype(jnp.int32),
              positive_sample.astype(jnp.int32),
              negative_sample.astype(jnp.int32))


# speedup vs baseline: 6.4656x; 1.1811x over previous
import functools

import jax
import jax.numpy as jnp
from jax import lax
from jax.experimental import pallas as pl
from jax.experimental import layout as jex_layout
from jax.experimental.pallas import tpu as pltpu
from jax.experimental.pallas import tpu_sc as plsc

D = 64
B = 4096
K = 128
GAMMA = 24.0
ALPHA = 0.02
NC = 2
NS = 16
NW = NC * NS
BQ = B // NW
L = 16
NG = K // L
QG = BQ // L
DG = D // L


def _sc_body(ent, off_t, ans, tm_t, ta_t, sm_t, sa_t, w_hbm,
             hid_hbm, rid_hbm, pid_hbm, nid_hbm, out_hbm,
             hid_v, rid_v, pid_v, nid_v, w_v,
             cen_v, box_v, tm_v, ta_v, sm_v, sa_v, pr_v,
             na_v, nb_v, nc_v, nd_v, out_v, sem0, sem1, sem2, sem3):
  wid = lax.axis_index("s") * NC + lax.axis_index("c")
  base = wid * BQ

  pltpu.sync_copy(hid_hbm.at[pl.ds(base, BQ)], hid_v)
  pltpu.sync_copy(rid_hbm.at[pl.ds(base, BQ)], rid_v)
  pltpu.sync_copy(pid_hbm.at[pl.ds(base, BQ)], pid_v)
  pltpu.sync_copy(w_hbm.at[pl.ds(base, BQ)], w_v)
  pltpu.sync_copy(nid_hbm.at[pl.ds(base, BQ)], nid_v)

  d0 = pltpu.async_copy(ent.at[hid_v], cen_v, sem0)
  d1 = pltpu.async_copy(off_t.at[hid_v], box_v, sem0)
  d2 = pltpu.async_copy(tm_t.at[rid_v], tm_v, sem0)
  d3 = pltpu.async_copy(ta_t.at[rid_v], ta_v, sem0)
  d4 = pltpu.async_copy(sm_t.at[rid_v], sm_v, sem0)
  d5 = pltpu.async_copy(sa_t.at[rid_v], sa_v, sem0)
  d6 = pltpu.async_copy(ans.at[pid_v], pr_v, sem0)
  for d in (d0, d1, d2, d3, d4, d5, d6):
    d.wait()

  def _mk_query(b, _):
    for g in range(DG):
      sl = pl.ds(g * L, L)
      e16 = cen_v[b, sl]
      cen_v[b, sl] = e16 * tm_v[b, sl] + ta_v[b, sl]
      o16 = box_v[b, sl]
      box_v[b, sl] = jnp.abs(o16 * sm_v[b, sl] + sa_v[b, sl])
    return 0

  lax.fori_loop(0, BQ, _mk_query, 0)

  iota = lax.iota(jnp.int32, L)
  one_m_alpha = jnp.float32(1.0 - ALPHA)
  zero16 = jnp.zeros((L,), jnp.float32)


  for g in range(QG):
    rowv = iota + g * L
    wv = w_v[pl.ds(g * L, L)]

    def _pos_d(d, carry):
      s1, s2 = carry
      dmask = jnp.bitwise_and(d + iota, D - 1)
      a = plsc.load_gather(pr_v, [rowv, dmask])
      c = plsc.load_gather(cen_v, [rowv, dmask])
      o = plsc.load_gather(box_v, [rowv, dmask])
      dv = jnp.abs(a - c)
      return s1 + dv, s2 + jnp.minimum(dv, o)

    s1, s2 = lax.fori_loop(0, D, _pos_d, (zero16, zero16))
    logit = (GAMMA - s1 + one_m_alpha * s2) * wv
    plsc.store_scatter(out_v, [rowv, jnp.zeros((L,), jnp.int32)], logit)

  def _neg_compute(b, buf):
    bsplat = jnp.full((L,), b, jnp.int32)
    wspl = plsc.load_gather(w_v, [bsplat])

    def _neg_d(d, carry):
      dmask = jnp.bitwise_and(d + iota, D - 1)
      c = plsc.load_gather(cen_v, [bsplat, dmask])
      o = plsc.load_gather(box_v, [bsplat, dmask])
      new = []
      for g in range(NG):
        s1, s2 = carry[2 * g], carry[2 * g + 1]
        a = plsc.load_gather(buf, [iota + g * L, dmask])
        dv = jnp.abs(a - c)
        new.append(s1 + dv)
        new.append(s2 + jnp.minimum(dv, o))
      return tuple(new)

    acc = lax.fori_loop(0, D, _neg_d, (zero16,) * (2 * NG))
    for g in range(NG):
      s1, s2 = acc[2 * g], acc[2 * g + 1]
      logit = (GAMMA - s1 + one_m_alpha * s2) * wspl
      cols = jnp.full((L,), 1 + g * L, jnp.int32) + iota
      plsc.store_scatter(out_v, [bsplat, cols], logit)

  bufs = (na_v, nb_v, nc_v, nd_v)
  sems = (sem0, sem1, sem2, sem3)
  for u in range(3):
    pltpu.async_copy(ans.at[nid_v.at[u]], bufs[u], sems[u])

  def _quad(i, _):
    b0 = 4 * i
    for u in range(4):
      b = b0 + u
      pltpu.make_async_copy(ans.at[nid_v.at[b]], bufs[u], sems[u]).wait()

      @pl.when(b + 3 < BQ)
      def _():
        pltpu.async_copy(ans.at[nid_v.at[b + 3]], bufs[(u + 3) % 4],
                         sems[(u + 3) % 4])

      _neg_compute(b, bufs[u])
    return 0

  lax.fori_loop(0, BQ // 4, _quad, 0)

  pltpu.sync_copy(out_v, out_hbm.at[pl.ds(base, BQ)])


@jax.jit
def _run(ent, off_t, ans, tm_t, ta_t, sm_t, sa_t, w, hid, rid, pid, nid):
  mesh = plsc.VectorSubcoreMesh(core_axis_name="c", subcore_axis_name="s")
  f = functools.partial(
      pl.kernel,
      out_type=jax.ShapeDtypeStruct((B, 1 + K), jnp.float32),
      mesh=mesh,
      compiler_params=pltpu.CompilerParams(
          needs_layout_passes=False, use_tc_tiling_on_sc=False),
      scratch_types=[
          pltpu.VMEM((BQ,), jnp.int32),
          pltpu.VMEM((BQ,), jnp.int32),
          pltpu.VMEM((BQ,), jnp.int32),
          pltpu.VMEM((BQ, K), jnp.int32),
          pltpu.VMEM((BQ,), jnp.float32),
          pltpu.VMEM((BQ, D), jnp.float32),
          pltpu.VMEM((BQ, D), jnp.float32),
          pltpu.VMEM((BQ, D), jnp.float32),
          pltpu.VMEM((BQ, D), jnp.float32),
          pltpu.VMEM((BQ, D), jnp.float32),
          pltpu.VMEM((BQ, D), jnp.float32),
          pltpu.VMEM((BQ, D), jnp.float32),
          pltpu.VMEM((K, D), jnp.float32),
          pltpu.VMEM((K, D), jnp.float32),
          pltpu.VMEM((K, D), jnp.float32),
          pltpu.VMEM((K, D), jnp.float32),
          pltpu.VMEM((BQ, 1 + K), jnp.float32),
          pltpu.SemaphoreType.DMA,
          pltpu.SemaphoreType.DMA,
          pltpu.SemaphoreType.DMA,
          pltpu.SemaphoreType.DMA,
      ],
  )(_sc_body)
  return f(ent, off_t, ans, tm_t, ta_t, sm_t, sa_t, w, hid, rid, pid, nid)


_ROWMAJOR_T8 = jex_layout.Layout(major_to_minor=(0, 1), tiling=((8,),))


def kernel(entity_embedding, offset_embedding, answer_embedding,
           translation_mul, translation_add, scaling_mul, scaling_add,
           subsampling_weight, head_ids, rel_ids, positive_sample,
           negative_sample):
  return _run(entity_embedding, offset_embedding, answer_embedding,
              translation_mul, translation_add, scaling_mul, scaling_add,
              subsampling_weight,
              head_ids.astype(jnp.int32), rel_ids.astype(jnp.int32),
              positive_sample.astype(jnp.int32),
              negative_sample.astype(jnp.int32))


# speedup vs baseline: 6.4702x; 1.0007x over previous
import functools

import jax
import jax.numpy as jnp
from jax import lax
from jax.experimental import pallas as pl
from jax.experimental.pallas import tpu as pltpu
from jax.experimental.pallas import tpu_sc as plsc

D = 64
B = 4096
K = 128
GAMMA = 24.0
ALPHA = 0.02
NC = 2
NS = 16
NW = NC * NS
BQ = B // NW
L = 16
NG = K // L
QG = BQ // L
DG = D // L


def _sc_body(ent, off_t, ans, tm_t, ta_t, sm_t, sa_t, w_hbm,
             hid_hbm, rid_hbm, pid_hbm, nid_hbm, out_hbm,
             hid_v, rid_v, pid_v, nid_v, w_v,
             cen_v, box_v, tm_v, ta_v, sm_v, sa_v, pr_v,
             na_v, nb_v, nc_v, nd_v, out_v, sem0, sem1, sem2, sem3):
  wid = lax.axis_index("s") * NC + lax.axis_index("c")
  base = wid * BQ

  pltpu.sync_copy(hid_hbm.at[pl.ds(base, BQ)], hid_v)
  pltpu.sync_copy(rid_hbm.at[pl.ds(base, BQ)], rid_v)
  pltpu.sync_copy(pid_hbm.at[pl.ds(base, BQ)], pid_v)
  pltpu.sync_copy(w_hbm.at[pl.ds(base, BQ)], w_v)
  pltpu.sync_copy(nid_hbm.at[pl.ds(base, BQ)], nid_v)

  d0 = pltpu.async_copy(ent.at[hid_v], cen_v, sem0)
  d1 = pltpu.async_copy(off_t.at[hid_v], box_v, sem0)
  d2 = pltpu.async_copy(tm_t.at[rid_v], tm_v, sem0)
  d3 = pltpu.async_copy(ta_t.at[rid_v], ta_v, sem0)
  d4 = pltpu.async_copy(sm_t.at[rid_v], sm_v, sem0)
  d5 = pltpu.async_copy(sa_t.at[rid_v], sa_v, sem0)
  d6 = pltpu.async_copy(ans.at[pid_v], pr_v, sem0)
  for d in (d0, d1, d2, d3, d4, d5, d6):
    d.wait()

  def _mk_query(b, _):
    for g in range(DG):
      sl = pl.ds(g * L, L)
      e16 = cen_v[b, sl]
      cen_v[b, sl] = e16 * tm_v[b, sl] + ta_v[b, sl]
      o16 = box_v[b, sl]
      box_v[b, sl] = jnp.abs(o16 * sm_v[b, sl] + sa_v[b, sl])
    return 0

  lax.fori_loop(0, BQ, _mk_query, 0)

  iota = lax.iota(jnp.int32, L)
  one_m_alpha = jnp.float32(1.0 - ALPHA)
  zero16 = jnp.zeros((L,), jnp.float32)


  for g in range(QG):
    rowv = iota + g * L
    wv = w_v[pl.ds(g * L, L)]

    def _pos_d(d, carry):
      s1, s2 = carry
      dmask = jnp.bitwise_and(d + iota, D - 1)
      a = plsc.load_gather(pr_v, [rowv, dmask])
      c = plsc.load_gather(cen_v, [rowv, dmask])
      o = plsc.load_gather(box_v, [rowv, dmask])
      dv = jnp.abs(a - c)
      return s1 + dv, s2 + jnp.minimum(dv, o)

    s1, s2 = lax.fori_loop(0, D, _pos_d, (zero16, zero16))
    logit = (GAMMA - s1 + one_m_alpha * s2) * wv
    plsc.store_scatter(out_v, [rowv, jnp.zeros((L,), jnp.int32)], logit)

  def _neg_compute(b, buf):
    bsplat = jnp.full((L,), b, jnp.int32)
    wspl = plsc.load_gather(w_v, [bsplat])

    def _neg_d(d, carry):
      dmask = jnp.bitwise_and(d + iota, D - 1)
      c = plsc.load_gather(cen_v, [bsplat, dmask])
      o = plsc.load_gather(box_v, [bsplat, dmask])
      new = []
      for g in range(NG):
        s1, s2 = carry[2 * g], carry[2 * g + 1]
        a = plsc.load_gather(buf, [iota + g * L, dmask])
        dv = jnp.abs(a - c)
        new.append(s1 + dv)
        new.append(s2 + jnp.minimum(dv, o))
      return tuple(new)

    acc = lax.fori_loop(0, D, _neg_d, (zero16,) * (2 * NG))
    for g in range(NG):
      s1, s2 = acc[2 * g], acc[2 * g + 1]
      logit = (GAMMA - s1 + one_m_alpha * s2) * wspl
      cols = jnp.full((L,), 1 + g * L, jnp.int32) + iota
      plsc.store_scatter(out_v, [bsplat, cols], logit)

  bufs = (na_v, nb_v, nc_v, nd_v)
  sems = (sem0, sem1, sem2, sem3)
  for u in range(3):
    pltpu.async_copy(ans.at[nid_v.at[u]], bufs[u], sems[u])

  def _quad(i, _):
    b0 = 4 * i
    for u in range(4):
      b = b0 + u
      pltpu.make_async_copy(ans.at[nid_v.at[b]], bufs[u], sems[u]).wait()

      @pl.when(b + 3 < BQ)
      def _():
        pltpu.async_copy(ans.at[nid_v.at[b + 3]], bufs[(u + 3) % 4],
                         sems[(u + 3) % 4])

      _neg_compute(b, bufs[u])
    return 0

  lax.fori_loop(0, BQ // 4, _quad, 0)

  pltpu.sync_copy(out_v, out_hbm.at[pl.ds(base, BQ)])


@jax.jit
def _run(ent, off_t, ans, tm_t, ta_t, sm_t, sa_t, w, hid, rid, pid, nid):
  mesh = plsc.VectorSubcoreMesh(core_axis_name="c", subcore_axis_name="s")
  f = functools.partial(
      pl.kernel,
      out_type=jax.ShapeDtypeStruct((B, 1 + K), jnp.float32),
      mesh=mesh,
      compiler_params=pltpu.CompilerParams(
          needs_layout_passes=False, use_tc_tiling_on_sc=False),
      scratch_types=[
          pltpu.VMEM((BQ,), jnp.int32),
          pltpu.VMEM((BQ,), jnp.int32),
          pltpu.VMEM((BQ,), jnp.int32),
          pltpu.VMEM((BQ, K), jnp.int32),
          pltpu.VMEM((BQ,), jnp.float32),
          pltpu.VMEM((BQ, D), jnp.float32),
          pltpu.VMEM((BQ, D), jnp.float32),
          pltpu.VMEM((BQ, D), jnp.float32),
          pltpu.VMEM((BQ, D), jnp.float32),
          pltpu.VMEM((BQ, D), jnp.float32),
          pltpu.VMEM((BQ, D), jnp.float32),
          pltpu.VMEM((BQ, D), jnp.float32),
          pltpu.VMEM((K, D), jnp.float32),
          pltpu.VMEM((K, D), jnp.float32),
          pltpu.VMEM((K, D), jnp.float32),
          pltpu.VMEM((K, D), jnp.float32),
          pltpu.VMEM((BQ, 1 + K), jnp.float32),
          pltpu.SemaphoreType.DMA,
          pltpu.SemaphoreType.DMA,
          pltpu.SemaphoreType.DMA,
          pltpu.SemaphoreType.DMA,
      ],
  )(_sc_body)
  return f(ent, off_t, ans, tm_t, ta_t, sm_t, sa_t, w, hid, rid, pid, nid)


def kernel(entity_embedding, offset_embedding, answer_embedding,
           translation_mul, translation_add, scaling_mul, scaling_add,
           subsampling_weight, head_ids, rel_ids, positive_sample,
           negative_sample):
  return _run(entity_embedding, offset_embedding, answer_embedding,
              translation_mul, translation_add, scaling_mul, scaling_add,
              subsampling_weight,
              head_ids.astype(jnp.int32), rel_ids.astype(jnp.int32),
              positive_sample.astype(jnp.int32),
              negative_sample.astype(jnp.int32))
